# baseline (device time: 33043 ns/iter reference)
import jax
import jax.numpy as jnp
from jax import lax
from jax.experimental import pallas as pl
from jax.experimental.pallas import tpu as pltpu

N_DEV = 4


def kernel(A, B):
    m_per, k = A.shape
    k2, n = B.shape
    assert k == k2
    m_half = m_per // 2
    m_q = m_per // 4
    TOP = pl.ds(0, m_half)
    BOT = pl.ds(m_half, m_half)
    QTR = [pl.ds(i * m_q, m_q) for i in range(4)]

    def body(
        a_ref, b_ref, out_ref,
        a_vmem, b_vmem, my_q, my_s, b_bf,
        recv_lq, recv_ls, recv_rq, recv_rs, recv_dq, recv_ds, out_vmem,
        send_sems, recv_sems, copy_sems, in_sems,
    ):
        my_pos = lax.axis_index("i")
        left = (my_pos - 1) % N_DEV
        right = (my_pos + 1) % N_DEV

        a_in = pltpu.make_async_copy(a_ref, a_vmem, in_sems.at[0])
        b_in = pltpu.make_async_copy(b_ref, b_vmem, in_sems.at[1])
        a_in.start()
        b_in.start()

        barrier_sem = pltpu.get_barrier_semaphore()
        for nbr in [left, right]:
            pl.semaphore_signal(
                barrier_sem, inc=1,
                device_id=(nbr,), device_id_type=pl.DeviceIdType.MESH,
            )

        a_in.wait()
        a = a_vmem[:, :]
        amax = jnp.maximum(
            jnp.max(jnp.abs(a), axis=1, keepdims=True), 1e-20
        )
        my_s[:, :] = amax * (1.0 / 127.0)
        my_q[:, :] = jnp.clip(
            jnp.round(a * (127.0 / amax)), -127.0, 127.0
        ).astype(jnp.int8)
        pl.semaphore_wait(barrier_sem, 2)

        def rdma(i, src, dst, dev):
            return pltpu.make_async_remote_copy(
                src_ref=src, dst_ref=dst,
                send_sem=send_sems.at[i], recv_sem=recv_sems.at[i],
                device_id=(dev,), device_id_type=pl.DeviceIdType.MESH,
            )

        p1 = [
            rdma(0, my_s.at[TOP, :], recv_ls.at[TOP, :], right),
            rdma(1, my_q.at[TOP, :], recv_lq.at[TOP, :], right),
            rdma(2, my_s.at[BOT, :], recv_rs.at[BOT, :], left),
            rdma(3, my_q.at[BOT, :], recv_rq.at[BOT, :], left),
            rdma(4, my_s.at[BOT, :], recv_ls.at[BOT, :], right),
            rdma(5, my_q.at[BOT, :], recv_lq.at[BOT, :], right),
            rdma(6, my_s.at[TOP, :], recv_rs.at[TOP, :], left),
            rdma(7, my_q.at[TOP, :], recv_rq.at[TOP, :], left),
        ]
        for r in p1:
            r.start()
        (s_rt_s, s_rt_q, s_lb_s, s_lb_q, s_rb_s, s_rb_q, s_lt_s, s_lt_q) = p1

        b_in.wait()
        b_bf[:, :] = b_vmem[:, :].astype(jnp.bfloat16)

        def mm_q(slot, rows, q_ref, s_ref):
            deq = (
                q_ref[rows, :].astype(jnp.float32) * s_ref[rows, :]
            ).astype(jnp.bfloat16)
            out_vmem[slot, rows, :] = jnp.dot(
                deq, b_bf[:, :], preferred_element_type=jnp.float32
            )

        def store(sem_i, slot, rows, origin, row_off, nrows):
            copy = pltpu.make_async_copy(
                out_vmem.at[slot, rows, :],
                out_ref.at[pl.ds(origin * m_per + row_off, nrows), :],
                copy_sems.at[sem_i],
            )
            copy.start()
            return copy

        out_vmem[0, TOP, :] = jnp.dot(
            a_vmem[TOP, :].astype(jnp.bfloat16), b_bf[:, :],
            preferred_element_type=jnp.float32,
        )
        c0 = store(0, 0, TOP, my_pos, 0, m_half)
        out_vmem[0, BOT, :] = jnp.dot(
            a_vmem[BOT, :].astype(jnp.bfloat16), b_bf[:, :],
            preferred_element_type=jnp.float32,
        )
        c1 = store(1, 0, BOT, my_pos, m_half, m_half)

        s_rt_s.wait_recv()
        s_rt_q.wait_recv()
        fwd_r = [
            rdma(8, recv_ls.at[QTR[0], :], recv_ds.at[QTR[0], :], right),
            rdma(9, recv_lq.at[QTR[0], :], recv_dq.at[QTR[0], :], right),
            rdma(10, recv_ls.at[QTR[1], :], recv_ds.at[QTR[1], :], right),
            rdma(11, recv_lq.at[QTR[1], :], recv_dq.at[QTR[1], :], right),
        ]
        for r in fwd_r:
            r.start()
        mm_q(1, TOP, recv_lq, recv_ls)
        c2 = store(2, 1, TOP, left, 0, m_half)

        s_lb_s.wait_recv()
        s_lb_q.wait_recv()
        fwd_l = [
            rdma(12, recv_rs.at[QTR[2], :], recv_ds.at[QTR[2], :], left),
            rdma(13, recv_rq.at[QTR[2], :], recv_dq.at[QTR[2], :], left),
            rdma(14, recv_rs.at[QTR[3], :], recv_ds.at[QTR[3], :], left),
            rdma(15, recv_rq.at[QTR[3], :], recv_dq.at[QTR[3], :], left),
        ]
        for r in fwd_l:
            r.start()
        mm_q(2, BOT, recv_rq, recv_rs)
        c3 = store(3, 2, BOT, right, m_half, m_half)

        s_rb_s.wait_recv()
        s_rb_q.wait_recv()
        mm_q(1, BOT, recv_lq, recv_ls)
        c4 = store(4, 1, BOT, left, m_half, m_half)
        s_lt_s.wait_recv()
        s_lt_q.wait_recv()
        mm_q(2, TOP, recv_rq, recv_rs)
        c5 = store(5, 2, TOP, right, 0, m_half)

        diag = (my_pos + 2) % N_DEV
        cq = []
        for qi, (s_d, q_d) in enumerate(
            [(fwd_r[0], fwd_r[1]), (fwd_r[2], fwd_r[3]),
             (fwd_l[0], fwd_l[1]), (fwd_l[2], fwd_l[3])]
        ):
            s_d.wait_recv()
            q_d.wait_recv()
            mm_q(3, QTR[qi], recv_dq, recv_ds)
            cq.append(store(6 + qi, 3, QTR[qi], diag, qi * m_q, m_q))

        for c in [c0, c1, c2, c3, c4, c5] + cq:
            c.wait()
        for s in p1 + fwd_r + fwd_l:
            s.wait_send()

    return pl.pallas_call(
        body,
        out_shape=jax.ShapeDtypeStruct((N_DEV * m_per, n), jnp.float32),
        in_specs=[
            pl.BlockSpec(memory_space=pl.ANY),
            pl.BlockSpec(memory_space=pl.ANY),
        ],
        out_specs=pl.BlockSpec(memory_space=pl.ANY),
        scratch_shapes=[
            pltpu.VMEM((m_per, k), jnp.float32),
            pltpu.VMEM((k, n), jnp.float32),
            pltpu.VMEM((m_per, k), jnp.int8),
            pltpu.VMEM((m_per, 1), jnp.float32),
            pltpu.VMEM((k, n), jnp.bfloat16),
            pltpu.VMEM((m_per, k), jnp.int8),
            pltpu.VMEM((m_per, 1), jnp.float32),
            pltpu.VMEM((m_per, k), jnp.int8),
            pltpu.VMEM((m_per, 1), jnp.float32),
            pltpu.VMEM((m_per, k), jnp.int8),
            pltpu.VMEM((m_per, 1), jnp.float32),
            pltpu.VMEM((N_DEV, m_per, n), jnp.float32),
            pltpu.SemaphoreType.DMA((16,)),
            pltpu.SemaphoreType.DMA((16,)),
            pltpu.SemaphoreType.DMA((10,)),
            pltpu.SemaphoreType.DMA((2,)),
        ],
        compiler_params=pltpu.CompilerParams(collective_id=0),
    )(A, B)


# device time: 32495 ns/iter; 1.0169x vs baseline; 1.0169x over previous
import jax
import jax.numpy as jnp
from jax import lax
from jax.experimental import pallas as pl
from jax.experimental.pallas import tpu as pltpu

N_DEV = 4


def kernel(A, B):
    m_per, k = A.shape
    k2, n = B.shape
    assert k == k2
    m_half = m_per // 2
    TOP = pl.ds(0, m_half)
    BOT = pl.ds(m_half, m_half)

    def body(
        a_ref, b_ref, out_ref,
        my_bf, b_bf, recv_l, recv_r, recv_d, out_vmem,
        send_sems, recv_sems, copy_sems,
    ):
        my_pos = lax.axis_index("i")
        left = (my_pos - 1) % N_DEV
        right = (my_pos + 1) % N_DEV

        barrier_sem = pltpu.get_barrier_semaphore()
        for nbr in [left, right]:
            pl.semaphore_signal(
                barrier_sem, inc=1,
                device_id=(nbr,), device_id_type=pl.DeviceIdType.MESH,
            )
        my_bf[:, :] = a_ref[:, :].astype(jnp.bfloat16)
        pl.semaphore_wait(barrier_sem, 2)

        def rdma(i, src, dst, dev):
            return pltpu.make_async_remote_copy(
                src_ref=src, dst_ref=dst,
                send_sem=send_sems.at[i], recv_sem=recv_sems.at[i],
                device_id=(dev,), device_id_type=pl.DeviceIdType.MESH,
            )

        s_rt = rdma(0, my_bf.at[TOP, :], recv_l.at[TOP, :], right)
        s_lb = rdma(1, my_bf.at[BOT, :], recv_r.at[BOT, :], left)
        s_rb = rdma(2, my_bf.at[BOT, :], recv_l.at[BOT, :], right)
        s_lt = rdma(3, my_bf.at[TOP, :], recv_r.at[TOP, :], left)
        s_rt.start()
        s_lb.start()
        s_rb.start()
        s_lt.start()

        b_bf[:, :] = b_ref[:, :].astype(jnp.bfloat16)

        def mm(slot, rows, chunk_rows):
            out_vmem[slot, rows, :] = jnp.dot(
                chunk_rows, b_bf[:, :], preferred_element_type=jnp.float32
            )

        def store_half(sem_i, slot, rows, origin, row_off):
            copy = pltpu.make_async_copy(
                out_vmem.at[slot, rows, :],
                out_ref.at[pl.ds(origin * m_per + row_off, m_half), :],
                copy_sems.at[sem_i],
            )
            copy.start()
            return copy

        mm(0, TOP, my_bf[TOP, :])
        c0 = store_half(0, 0, TOP, my_pos, 0)
        mm(0, BOT, my_bf[BOT, :])
        c1 = store_half(1, 0, BOT, my_pos, m_half)

        s_rt.wait_recv()
        f_r = rdma(4, recv_l.at[TOP, :], recv_d.at[TOP, :], right)
        f_r.start()
        mm(1, TOP, recv_l[TOP, :])
        c2 = store_half(2, 1, TOP, left, 0)

        s_lb.wait_recv()
        f_l = rdma(5, recv_r.at[BOT, :], recv_d.at[BOT, :], left)
        f_l.start()
        mm(2, BOT, recv_r[BOT, :])
        c3 = store_half(3, 2, BOT, right, m_half)

        s_rb.wait_recv()
        mm(1, BOT, recv_l[BOT, :])
        c4 = store_half(4, 1, BOT, left, m_half)
        s_lt.wait_recv()
        mm(2, TOP, recv_r[TOP, :])
        c5 = store_half(5, 2, TOP, right, 0)

        diag = (my_pos + 2) % N_DEV
        f_r.wait_recv()
        mm(3, TOP, recv_d[TOP, :])
        c6 = store_half(6, 3, TOP, diag, 0)
        f_l.wait_recv()
        mm(3, BOT, recv_d[BOT, :])
        c7 = store_half(7, 3, BOT, diag, m_half)

        for c in [c0, c1, c2, c3, c4, c5, c6, c7]:
            c.wait()
        for s in [s_rt, s_lb, s_rb, s_lt, f_r, f_l]:
            s.wait_send()

    return pl.pallas_call(
        body,
        out_shape=jax.ShapeDtypeStruct((N_DEV * m_per, n), jnp.float32),
        in_specs=[
            pl.BlockSpec(memory_space=pltpu.VMEM),
            pl.BlockSpec(memory_space=pltpu.VMEM),
        ],
        out_specs=pl.BlockSpec(memory_space=pl.ANY),
        scratch_shapes=[
            pltpu.VMEM((m_per, k), jnp.bfloat16),
            pltpu.VMEM((k, n), jnp.bfloat16),
            pltpu.VMEM((m_per, k), jnp.bfloat16),
            pltpu.VMEM((m_per, k), jnp.bfloat16),
            pltpu.VMEM((m_per, k), jnp.bfloat16),
            pltpu.VMEM((N_DEV, m_per, n), jnp.float32),
            pltpu.SemaphoreType.DMA((6,)),
            pltpu.SemaphoreType.DMA((6,)),
            pltpu.SemaphoreType.DMA((8,)),
        ],
        compiler_params=pltpu.CompilerParams(collective_id=0),
    )(A, B)
